# baseline (speedup 1.0000x reference)
import functools

import jax
import jax.numpy as jnp
from jax import lax
from jax.experimental import pallas as pl
from jax.experimental.pallas import tpu as pltpu
from jax.experimental.pallas import tpu_sc as plsc

N = 10000
E = 160000
H = 128
CHUNK = 64
NTILES = 32
NCHUNKS = E // CHUNK
CLOOP = 80
NPAD = 10112
ROWS_PER_SUB = NPAD // 16

_SC_MESH = plsc.VectorSubcoreMesh(core_axis_name="c", subcore_axis_name="s")

H2 = H // 2

_LO_IDX = tuple(32 * (j // 16) + j % 16 for j in range(H2))
_HI_IDX = tuple(32 * (j // 16) + 16 + j % 16 for j in range(H2))


def _sc_msg_body(h_hbm, hb_hbm, src_hbm, dst_hbm, zero_hbm, out_hbm,
                 s0, s1, s2, s3, d0, d1, d2, d3, r0, r1, r2, r3, b0, b1,
                 acc_sh,
                 si0, si1, si2, si3, sg0, sg1, sh0, sh1, ss0, ss1):
    srcs = [s0, s1, s2, s3]
    dsts = [d0, d1, d2, d3]
    rows = [r0, r1, r2, r3]
    hbb = [b0, b1]
    sem_i = [si0, si1, si2, si3]
    sem_g = [sg0, sg1]
    sem_h = [sh0, sh1]
    sem_s = [ss0, ss1]

    c = lax.axis_index("c")
    s = lax.axis_index("s")
    wid = s * 2 + c

    pltpu.sync_copy(zero_hbm, acc_sh.at[pl.ds(s * ROWS_PER_SUB, ROWS_PER_SUB)])
    plsc.subcore_barrier()

    def _cid(j):
        return wid + j * NTILES

    def _ok(j):
        return jnp.logical_and(j >= 0, _cid(j) < NCHUNKS)

    def issue_idx(j, m):
        @pl.when(_ok(j))
        def _():
            base = _cid(j) * CHUNK
            pltpu.async_copy(src_hbm.at[pl.ds(base, CHUNK)], srcs[m], sem_i[m])
            pltpu.async_copy(dst_hbm.at[pl.ds(base, CHUNK)], dsts[m], sem_i[m])

    def wait_idx(j, m):
        @pl.when(_ok(j))
        def _():
            base = _cid(j) * CHUNK
            pltpu.make_async_copy(src_hbm.at[pl.ds(base, CHUNK)], srcs[m], sem_i[m]).wait()
            pltpu.make_async_copy(dst_hbm.at[pl.ds(base, CHUNK)], dsts[m], sem_i[m]).wait()

    def issue_fetch(j, m, p):
        @pl.when(_ok(j))
        def _():
            base = _cid(j) * CHUNK
            pltpu.async_copy(h_hbm.at[srcs[m]], rows[m], sem_g[p])
            pltpu.async_copy(hb_hbm.at[pl.ds(base, CHUNK)], hbb[p], sem_h[p])

    def wait_fetch(j, m, p):
        @pl.when(_ok(j))
        def _():
            base = _cid(j) * CHUNK
            pltpu.make_async_copy(h_hbm.at[srcs[m]], rows[m], sem_g[p]).wait()
            pltpu.make_async_copy(hb_hbm.at[pl.ds(base, CHUNK)], hbb[p], sem_h[p]).wait()

    def compute(j, m, p):
        @pl.when(_ok(j))
        def _():
            def _row(r, rc):
                for q in range(8):
                    sl = pl.ds(q * 16, 16)
                    v = rows[m][r, sl] + hbb[p][r, sl]
                    rows[m][r, sl] = jnp.maximum(v, 0.0)
                return rc
            lax.fori_loop(0, CHUNK, _row, 0)

    def issue_scatter(j, m, p):
        @pl.when(_ok(j))
        def _():
            pltpu.async_copy(rows[m], acc_sh.at[dsts[m]], sem_s[p], add=True)

    def wait_scatter(j, m, p):
        @pl.when(_ok(j))
        def _():
            pltpu.make_async_copy(rows[m], acc_sh.at[dsts[m]], sem_s[p]).wait()

    issue_idx(0, 0)
    issue_idx(1, 1)
    wait_idx(0, 0)
    issue_fetch(0, 0, 0)

    def _outer(t, carry):
        for r in range(4):
            k = 4 * t + r
            wait_scatter(k - 2, (r + 2) % 4, r % 2)
            wait_idx(k + 1, (r + 1) % 4)
            issue_fetch(k + 1, (r + 1) % 4, (r + 1) % 2)
            issue_idx(k + 2, (r + 2) % 4)
            wait_fetch(k, r % 4, r % 2)
            compute(k, r % 4, r % 2)
            issue_scatter(k, r % 4, r % 2)
        return carry

    lax.fori_loop(0, CLOOP // 4, _outer, 0)
    wait_scatter(CLOOP - 2, (CLOOP - 2) % 4, CLOOP % 2)
    wait_scatter(CLOOP - 1, (CLOOP - 1) % 4, (CLOOP + 1) % 2)

    plsc.subcore_barrier()
    pltpu.sync_copy(acc_sh.at[pl.ds(s * ROWS_PER_SUB, ROWS_PER_SUB)],
                    out_hbm.at[c, pl.ds(s * ROWS_PER_SUB, ROWS_PER_SUB)])


_sc_msg = functools.partial(
    pl.kernel,
    out_type=jax.ShapeDtypeStruct((2, NPAD, H), jnp.float32),
    mesh=_SC_MESH,
    scratch_types=(
        [pltpu.VMEM((CHUNK,), jnp.int32)] * 8
        + [pltpu.VMEM((CHUNK, H), jnp.float32)] * 6
        + [pltpu.VMEM_SHARED((NPAD, H), jnp.float32)]
        + [pltpu.SemaphoreType.DMA] * 10
    ),
)(_sc_msg_body)


def _sc_pair_body(h_hbm, src_hbm, dst_hbm, hp_hbm,
                  s0, s1, s2, s3, d0, d1, d2, d3,
                  rs0, rs1, rs2, rs3, rd0, rd1, rd2, rd3,
                  bp0, bp1, bp2, bp3,
                  si0, si1, si2, si3, sg0, sg1, sw0, sw1):
    srcs = [s0, s1, s2, s3]
    dsts = [d0, d1, d2, d3]
    rows_s = [rs0, rs1, rs2, rs3]
    rows_d = [rd0, rd1, rd2, rd3]
    bf_p = [bp0, bp1, bp2, bp3]
    sem_i = [si0, si1, si2, si3]
    sem_g = [sg0, sg1]
    sem_w = [sw0, sw1]

    c = lax.axis_index("c")
    s = lax.axis_index("s")
    wid = s * 2 + c

    def _cid(j):
        return wid + j * NTILES

    def _ok(j):
        return jnp.logical_and(j >= 0, _cid(j) < NCHUNKS)

    def issue_idx(j, m):
        @pl.when(_ok(j))
        def _():
            base = _cid(j) * CHUNK
            pltpu.async_copy(src_hbm.at[pl.ds(base, CHUNK)], srcs[m], sem_i[m])
            pltpu.async_copy(dst_hbm.at[pl.ds(base, CHUNK)], dsts[m], sem_i[m])

    def wait_idx(j, m):
        @pl.when(_ok(j))
        def _():
            base = _cid(j) * CHUNK
            pltpu.make_async_copy(src_hbm.at[pl.ds(base, CHUNK)], srcs[m], sem_i[m]).wait()
            pltpu.make_async_copy(dst_hbm.at[pl.ds(base, CHUNK)], dsts[m], sem_i[m]).wait()

    def issue_gather(j, m, p):
        @pl.when(_ok(j))
        def _():
            pltpu.async_copy(h_hbm.at[srcs[m]], rows_s[m], sem_g[p])
            pltpu.async_copy(h_hbm.at[dsts[m]], rows_d[m], sem_g[p])

    def wait_gather(j, m, p):
        @pl.when(_ok(j))
        def _():
            pltpu.make_async_copy(h_hbm.at[srcs[m]], rows_s[m], sem_g[p]).wait()
            pltpu.make_async_copy(h_hbm.at[dsts[m]], rows_d[m], sem_g[p]).wait()

    def compute(j, m):
        @pl.when(_ok(j))
        def _():
            def _bf16_word(x_ref, r, g):
                a = x_ref[r, pl.ds(g * 32, 16)]
                b = x_ref[r, pl.ds(g * 32 + 16, 16)]
                ra = a + 0x7FFF + ((a >> 16) & 1)
                rb = b + 0x7FFF + ((b >> 16) & 1)
                return ((ra >> 16) & 0xFFFF) | (rb & jnp.int32(-65536))

            def _row(r, rc):
                for g in range(4):
                    bf_p[m][r, pl.ds(g * 16, 16)] = _bf16_word(rows_s[m], r, g)
                    bf_p[m][r, pl.ds(H2 + g * 16, 16)] = _bf16_word(rows_d[m], r, g)
                return rc
            lax.fori_loop(0, CHUNK, _row, 0)

    def issue_write(j, m, p):
        @pl.when(_ok(j))
        def _():
            base = _cid(j) * CHUNK
            pltpu.async_copy(bf_p[m], hp_hbm.at[pl.ds(base, CHUNK)], sem_w[p])

    def wait_write(j, m, p):
        @pl.when(_ok(j))
        def _():
            base = _cid(j) * CHUNK
            pltpu.make_async_copy(bf_p[m], hp_hbm.at[pl.ds(base, CHUNK)], sem_w[p]).wait()

    issue_idx(0, 0)
    issue_idx(1, 1)
    wait_idx(0, 0)
    issue_gather(0, 0, 0)

    def _outer(t, carry):
        for r in range(4):
            k = 4 * t + r
            wait_write(k - 2, (r + 2) % 4, r % 2)
            wait_idx(k + 1, (r + 1) % 4)
            issue_gather(k + 1, (r + 1) % 4, (r + 1) % 2)
            issue_idx(k + 2, (r + 2) % 4)
            wait_gather(k, r % 4, r % 2)
            compute(k, r % 4)
            issue_write(k, r % 4, r % 2)
        return carry

    lax.fori_loop(0, CLOOP // 4, _outer, 0)
    wait_write(CLOOP - 2, (CLOOP - 2) % 4, CLOOP % 2)
    wait_write(CLOOP - 1, (CLOOP - 1) % 4, (CLOOP + 1) % 2)


_sc_pair = functools.partial(
    pl.kernel,
    out_type=jax.ShapeDtypeStruct((E, H), jnp.int32),
    mesh=_SC_MESH,
    scratch_types=(
        [pltpu.VMEM((CHUNK,), jnp.int32)] * 8
        + [pltpu.VMEM((CHUNK, H), jnp.int32)] * 12
        + [pltpu.SemaphoreType.DMA] * 8
    ),
)(_sc_pair_body)


def _mlp2_kern(x_ref, w1_ref, b1_ref, w2_ref, b2_ref, o_ref):
    t = jnp.maximum(
        jnp.dot(x_ref[...].astype(w1_ref.dtype), w1_ref[...],
                preferred_element_type=jnp.float32)
        + b1_ref[...], 0.0)
    o_ref[...] = (jnp.dot(t.astype(w2_ref.dtype), w2_ref[...],
                          preferred_element_type=jnp.float32)
                  + b2_ref[...])


def _mlp2(x, w1, b1, w2, b2, blk):
    m, din = x.shape
    dmid = w1.shape[1]
    dout = w2.shape[1]
    grid = m // blk
    return pl.pallas_call(
        _mlp2_kern,
        grid=(grid,),
        in_specs=[
            pl.BlockSpec((blk, din), lambda i: (i, 0)),
            pl.BlockSpec((din, dmid), lambda i: (0, 0)),
            pl.BlockSpec((1, dmid), lambda i: (0, 0)),
            pl.BlockSpec((dmid, dout), lambda i: (0, 0)),
            pl.BlockSpec((1, dout), lambda i: (0, 0)),
        ],
        out_specs=pl.BlockSpec((blk, dout), lambda i: (i, 0)),
        out_shape=jax.ShapeDtypeStruct((m, dout), jnp.float32),
    )(x, w1, b1.reshape(1, -1), w2, b2.reshape(1, -1))


def _bond_mlp_kern(x_ref, w1_ref, b1_ref, w2_ref, b2_ref, o_ref):
    for j in range(8):
        xj = x_ref[:, j * 16:(j + 1) * 16].astype(w1_ref.dtype)
        t = jnp.maximum(
            jnp.dot(xj, w1_ref[...], preferred_element_type=jnp.float32)
            + b1_ref[...], 0.0)
        o_ref[:, j * H:(j + 1) * H] = (
            jnp.dot(t.astype(w2_ref.dtype), w2_ref[...],
                    preferred_element_type=jnp.float32) + b2_ref[...])


def _bond_mlp(ea8, w1, b1, w2, b2, blk=2000):
    m = ea8.shape[0]
    grid = m // blk
    return pl.pallas_call(
        _bond_mlp_kern,
        grid=(grid,),
        in_specs=[
            pl.BlockSpec((blk, 128), lambda i: (i, 0)),
            pl.BlockSpec((16, 256), lambda i: (0, 0)),
            pl.BlockSpec((1, 256), lambda i: (0, 0)),
            pl.BlockSpec((256, H), lambda i: (0, 0)),
            pl.BlockSpec((1, H), lambda i: (0, 0)),
        ],
        out_specs=pl.BlockSpec((blk, 8 * H), lambda i: (i, 0)),
        out_shape=jax.ShapeDtypeStruct((m, 8 * H), jnp.float32),
    )(ea8, w1, b1.reshape(1, -1), w2, b2.reshape(1, -1))


def _bond_mlp_kern(xt_ref, w1_ref, b1_ref, w2_ref, b2_ref, o_ref):
    t = jnp.maximum(
        jax.lax.dot_general(
            xt_ref[...].astype(w1_ref.dtype), w1_ref[...],
            (((0,), (0,)), ((), ())), preferred_element_type=jnp.float32)
        + b1_ref[...], 0.0)
    o_ref[...] = (jnp.dot(t.astype(w2_ref.dtype), w2_ref[...],
                          preferred_element_type=jnp.float32) + b2_ref[...])


def _bond_mlp(eat, w1, b1, w2, b2, blk=16000):
    grid = E // blk
    din = eat.shape[0]
    dmid = w1.shape[1]
    dout = w2.shape[1]
    return pl.pallas_call(
        _bond_mlp_kern,
        grid=(grid,),
        in_specs=[
            pl.BlockSpec((din, blk), lambda i: (0, i)),
            pl.BlockSpec((din, dmid), lambda i: (0, 0)),
            pl.BlockSpec((1, dmid), lambda i: (0, 0)),
            pl.BlockSpec((dmid, dout), lambda i: (0, 0)),
            pl.BlockSpec((1, dout), lambda i: (0, 0)),
        ],
        out_specs=pl.BlockSpec((blk, dout), lambda i: (i, 0)),
        out_shape=jax.ShapeDtypeStruct((E, dout), jnp.float32),
    )(eat, w1, b1.reshape(1, -1), w2, b2.reshape(1, -1))


def _gru_kern(p_ref, h_ref, wih_ref, whh_ref, bg_ref, o_ref):
    msg = p_ref[0] + p_ref[1]
    gi = jnp.dot(msg.astype(wih_ref.dtype), wih_ref[...],
                 preferred_element_type=jnp.float32) + bg_ref[...]
    gh = jnp.dot(h_ref[...].astype(whh_ref.dtype), whh_ref[...],
                 preferred_element_type=jnp.float32)
    z = jax.nn.sigmoid(gi[:, :H] + gh[:, :H])
    r = jax.nn.sigmoid(gi[:, H:2 * H] + gh[:, H:2 * H])
    n = jnp.tanh(gi[:, 2 * H:] + r * gh[:, 2 * H:])
    o_ref[...] = (1.0 - z) * n + z * h_ref[...]


def _gru(partials, h, wih, whh, bg, blk=2000):
    grid = N // blk
    return pl.pallas_call(
        _gru_kern,
        grid=(grid,),
        in_specs=[
            pl.BlockSpec((2, blk, H), lambda i: (0, i, 0)),
            pl.BlockSpec((blk, H), lambda i: (i, 0)),
            pl.BlockSpec((H, 3 * H), lambda i: (0, 0)),
            pl.BlockSpec((H, 3 * H), lambda i: (0, 0)),
            pl.BlockSpec((1, 3 * H), lambda i: (0, 0)),
        ],
        out_specs=pl.BlockSpec((blk, H), lambda i: (i, 0)),
        out_shape=jax.ShapeDtypeStruct((N, H), jnp.float32),
    )(partials, h, wih, whh, bg.reshape(1, -1))


def _bond_out_kern(hp_ref, w1_ref, b1_ref,
                   w2_ref, b2_ref, w3_ref, b3_ref, o_ref):
    bf16 = jnp.bfloat16
    hp = hp_ref[...]

    def _unpk(x, idx):
        w = pltpu.unpack_elementwise(
            x, index=idx,
            packed_dtype=jnp.int16, unpacked_dtype=jnp.int32)
        return pltpu.bitcast(w << 16, jnp.float32).astype(bf16)

    xs, xd = hp[:, :H2], hp[:, H2:]
    ys = jnp.concatenate(
        [_unpk(xs, 0), _unpk(xs, 1), _unpk(xd, 0), _unpk(xd, 1)], axis=1)
    t = jnp.maximum(
        jnp.dot(ys, w1_ref[...], preferred_element_type=jnp.float32)
        + b1_ref[...], 0.0)
    t = jnp.maximum(
        jnp.dot(t.astype(w2_ref.dtype), w2_ref[...],
                preferred_element_type=jnp.float32)
        + b2_ref[...], 0.0)
    o_ref[...] = (jax.lax.dot_general(
        w3_ref[...], t.astype(w3_ref.dtype), (((0,), (1,)), ((), ())),
        preferred_element_type=jnp.float32) + b3_ref[...])


def _bond_out(hp, wo1, bo1, wo2, bo2, wo3, bo3, blk=16000):
    grid = E // blk
    dmid = wo1.shape[1]
    dmid2 = wo2.shape[1]
    k = 8
    lo = jnp.array(_LO_IDX)
    hi = jnp.array(_HI_IDX)
    w1 = jnp.concatenate(
        [wo1[:H][lo], wo1[:H][hi], wo1[H:][lo], wo1[H:][hi]], axis=0)
    return pl.pallas_call(
        _bond_out_kern,
        grid=(grid,),
        in_specs=[
            pl.BlockSpec((blk, H), lambda i: (i, 0)),
            pl.BlockSpec((2 * H, dmid), lambda i: (0, 0)),
            pl.BlockSpec((1, dmid), lambda i: (0, 0)),
            pl.BlockSpec((dmid, dmid2), lambda i: (0, 0)),
            pl.BlockSpec((1, dmid2), lambda i: (0, 0)),
            pl.BlockSpec((dmid2, k), lambda i: (0, 0)),
            pl.BlockSpec((k, 1), lambda i: (0, 0)),
        ],
        out_specs=pl.BlockSpec((k, blk), lambda i: (0, i)),
        out_shape=jax.ShapeDtypeStruct((k, E), jnp.float32),
    )(hp, w1,
      bo1.reshape(1, -1), wo2, bo2.reshape(1, -1),
      jnp.pad(wo3, ((0, 0), (0, 8 - wo3.shape[1]))),
      jnp.pad(bo3, (0, 8 - bo3.shape[0])).reshape(-1, 1))


def _graph_out_kern(h_ref, w1_ref, b1_ref, w2_ref, b2_ref, w3_ref, b3_ref,
                    o_ref):
    g = jnp.sum(h_ref[...], axis=0, keepdims=True) * (1.0 / N)
    t = jnp.maximum(
        jnp.dot(g, w1_ref[...], preferred_element_type=jnp.float32)
        + b1_ref[...], 0.0)
    t = jnp.maximum(
        jnp.dot(t, w2_ref[...], preferred_element_type=jnp.float32)
        + b2_ref[...], 0.0)
    o_ref[...] = (jnp.dot(t, w3_ref[...], preferred_element_type=jnp.float32)
                  + b3_ref[...])


def _graph_out(h, wq1, bq1, wq2, bq2, wq3, bq3):
    return pl.pallas_call(
        _graph_out_kern,
        out_shape=jax.ShapeDtypeStruct((1, wq3.shape[1]), jnp.float32),
    )(h, wq1, bq1.reshape(1, -1), wq2, bq2.reshape(1, -1), wq3,
      bq3.reshape(1, -1))


def kernel(x, edge_index, edge_attr, Wa1, ba1, Wa2, ba2, Wb1, bb1, Wb2, bb2,
           Wgih, Wghh, bg, Wo1, bo1, Wo2, bo2, Wo3, bo3, Wq1, bq1, Wq2, bq2,
           Wq3, bq3):
    src = edge_index[0]
    dst = edge_index[1]

    wa2p = jnp.pad(Wa2, ((0, 0), (0, 1)))
    ba2p = jnp.pad(ba2, (0, 1))
    h = _mlp2(x, Wa1, ba1, wa2p, ba2p, blk=2000)

    bf16 = jnp.bfloat16
    hb = _bond_mlp(edge_attr.T, Wb1.astype(bf16), bb1, Wb2.astype(bf16), bb2)

    zrows = jnp.zeros((ROWS_PER_SUB, H), jnp.float32)
    for _ in range(3):
        partials = _sc_msg(h, hb, src, dst, zrows)
        h = _gru(partials, h, Wgih.astype(bf16), Wghh.astype(bf16), bg)

    hp = _sc_pair(jax.lax.bitcast_convert_type(h, jnp.int32), src, dst)
    bond_t = _bond_out(hp, Wo1.astype(bf16), bo1, Wo2.astype(bf16),
                       bo2, Wo3.astype(bf16), bo3)
    bond_scores = bond_t[:Wo3.shape[1]].T

    graph_scores = _graph_out(h, Wq1, bq1, Wq2, bq2, Wq3, bq3)
    return bond_scores, graph_scores.reshape(-1)


# speedup vs baseline: 1.0173x; 1.0173x over previous
import functools

import jax
import jax.numpy as jnp
from jax import lax
from jax.experimental import pallas as pl
from jax.experimental.pallas import tpu as pltpu
from jax.experimental.pallas import tpu_sc as plsc

N = 10000
E = 160000
H = 128
CHUNK = 64
NTILES = 32
NCHUNKS = E // CHUNK
CLOOP = 80
NPAD = 10112
ROWS_PER_SUB = NPAD // 16

_SC_MESH = plsc.VectorSubcoreMesh(core_axis_name="c", subcore_axis_name="s")

H2 = H // 2

_LO_IDX = tuple(32 * (j // 16) + j % 16 for j in range(H2))
_HI_IDX = tuple(32 * (j // 16) + 16 + j % 16 for j in range(H2))


def _sc_msg_body(h_hbm, hb_hbm, src_hbm, dst_hbm, zero_hbm, out_hbm,
                 s0, s1, s2, s3, d0, d1, d2, d3, r0, r1, r2, r3, b0, b1,
                 acc_sh,
                 si0, si1, si2, si3, sd0, sd1, sd2, sd3,
                 sg0, sg1, sg2, sg3, sh0, sh1, ss0, ss1):
    srcs = [s0, s1, s2, s3]
    dsts = [d0, d1, d2, d3]
    rows = [r0, r1, r2, r3]
    hbb = [b0, b1]
    sem_i = [si0, si1, si2, si3]
    sem_di = [sd0, sd1, sd2, sd3]
    sem_g = [sg0, sg1, sg2, sg3]
    sem_h = [sh0, sh1]
    sem_s = [ss0, ss1]

    c = lax.axis_index("c")
    s = lax.axis_index("s")
    wid = s * 2 + c

    pltpu.sync_copy(zero_hbm, acc_sh.at[pl.ds(s * ROWS_PER_SUB, ROWS_PER_SUB)])
    plsc.subcore_barrier()

    def _cid(j):
        return wid + j * NTILES

    def _ok(j):
        return jnp.logical_and(j >= 0, _cid(j) < NCHUNKS)

    def issue_src_idx(j, m):
        @pl.when(_ok(j))
        def _():
            base = _cid(j) * CHUNK
            pltpu.async_copy(src_hbm.at[pl.ds(base, CHUNK)], srcs[m], sem_i[m])

    def wait_src_idx(j, m):
        @pl.when(_ok(j))
        def _():
            base = _cid(j) * CHUNK
            pltpu.make_async_copy(src_hbm.at[pl.ds(base, CHUNK)], srcs[m], sem_i[m]).wait()

    def issue_dst_idx(j, m):
        @pl.when(_ok(j))
        def _():
            base = _cid(j) * CHUNK
            pltpu.async_copy(dst_hbm.at[pl.ds(base, CHUNK)], dsts[m], sem_di[m])

    def wait_dst_idx(j, m):
        @pl.when(_ok(j))
        def _():
            base = _cid(j) * CHUNK
            pltpu.make_async_copy(dst_hbm.at[pl.ds(base, CHUNK)], dsts[m], sem_di[m]).wait()

    def issue_gather(j, m, p):
        @pl.when(_ok(j))
        def _():
            pltpu.async_copy(h_hbm.at[srcs[m]], rows[m], sem_g[p])

    def wait_gather(j, m, p):
        @pl.when(_ok(j))
        def _():
            pltpu.make_async_copy(h_hbm.at[srcs[m]], rows[m], sem_g[p]).wait()

    def issue_hb(j, p):
        @pl.when(_ok(j))
        def _():
            base = _cid(j) * CHUNK
            pltpu.async_copy(hb_hbm.at[pl.ds(base, CHUNK)], hbb[p], sem_h[p])

    def wait_hb(j, p):
        @pl.when(_ok(j))
        def _():
            base = _cid(j) * CHUNK
            pltpu.make_async_copy(hb_hbm.at[pl.ds(base, CHUNK)], hbb[p], sem_h[p]).wait()

    def compute(j, m, p):
        @pl.when(_ok(j))
        def _():
            def _row(r, rc):
                for q in range(8):
                    sl = pl.ds(q * 16, 16)
                    v = rows[m][r, sl] + hbb[p][r, sl]
                    rows[m][r, sl] = jnp.maximum(v, 0.0)
                return rc
            lax.fori_loop(0, CHUNK, _row, 0)

    def issue_scatter(j, m, p):
        @pl.when(_ok(j))
        def _():
            pltpu.async_copy(rows[m], acc_sh.at[dsts[m]], sem_s[p], add=True)

    def wait_scatter(j, m, p):
        @pl.when(_ok(j))
        def _():
            pltpu.make_async_copy(rows[m], acc_sh.at[dsts[m]], sem_s[p]).wait()

    issue_src_idx(0, 0)
    issue_src_idx(1, 1)
    issue_src_idx(2, 2)
    wait_src_idx(0, 0)
    issue_gather(0, 0, 0)
    wait_src_idx(1, 1)
    issue_gather(1, 1, 1)
    issue_dst_idx(0, 0)
    issue_hb(0, 0)

    def _outer(t, carry):
        for r in range(4):
            k = 4 * t + r
            wait_scatter(k - 2, (r + 2) % 4, r % 2)
            wait_src_idx(k + 2, (r + 2) % 4)
            issue_gather(k + 2, (r + 2) % 4, (r + 2) % 4)
            issue_src_idx(k + 3, (r + 3) % 4)
            issue_dst_idx(k + 1, (r + 1) % 4)
            issue_hb(k + 1, (r + 1) % 2)
            wait_gather(k, r % 4, r % 4)
            wait_hb(k, r % 2)
            wait_dst_idx(k, r % 4)
            compute(k, r % 4, r % 2)
            issue_scatter(k, r % 4, r % 2)
        return carry

    lax.fori_loop(0, CLOOP // 4, _outer, 0)
    wait_scatter(CLOOP - 2, (CLOOP - 2) % 4, CLOOP % 2)
    wait_scatter(CLOOP - 1, (CLOOP - 1) % 4, (CLOOP + 1) % 2)

    plsc.subcore_barrier()
    pltpu.sync_copy(acc_sh.at[pl.ds(s * ROWS_PER_SUB, ROWS_PER_SUB)],
                    out_hbm.at[c, pl.ds(s * ROWS_PER_SUB, ROWS_PER_SUB)])


_sc_msg = functools.partial(
    pl.kernel,
    out_type=jax.ShapeDtypeStruct((2, NPAD, H), jnp.float32),
    mesh=_SC_MESH,
    scratch_types=(
        [pltpu.VMEM((CHUNK,), jnp.int32)] * 8
        + [pltpu.VMEM((CHUNK, H), jnp.float32)] * 6
        + [pltpu.VMEM_SHARED((NPAD, H), jnp.float32)]
        + [pltpu.SemaphoreType.DMA] * 16
    ),
)(_sc_msg_body)


def _sc_pair_body(h_hbm, src_hbm, dst_hbm, hp_hbm,
                  s0, s1, s2, s3, d0, d1, d2, d3,
                  rs0, rs1, rs2, rs3, rd0, rd1, rd2, rd3,
                  bp0, bp1, bp2, bp3,
                  si0, si1, si2, si3, sg0, sg1, sw0, sw1):
    srcs = [s0, s1, s2, s3]
    dsts = [d0, d1, d2, d3]
    rows_s = [rs0, rs1, rs2, rs3]
    rows_d = [rd0, rd1, rd2, rd3]
    bf_p = [bp0, bp1, bp2, bp3]
    sem_i = [si0, si1, si2, si3]
    sem_g = [sg0, sg1]
    sem_w = [sw0, sw1]

    c = lax.axis_index("c")
    s = lax.axis_index("s")
    wid = s * 2 + c

    def _cid(j):
        return wid + j * NTILES

    def _ok(j):
        return jnp.logical_and(j >= 0, _cid(j) < NCHUNKS)

    def issue_idx(j, m):
        @pl.when(_ok(j))
        def _():
            base = _cid(j) * CHUNK
            pltpu.async_copy(src_hbm.at[pl.ds(base, CHUNK)], srcs[m], sem_i[m])
            pltpu.async_copy(dst_hbm.at[pl.ds(base, CHUNK)], dsts[m], sem_i[m])

    def wait_idx(j, m):
        @pl.when(_ok(j))
        def _():
            base = _cid(j) * CHUNK
            pltpu.make_async_copy(src_hbm.at[pl.ds(base, CHUNK)], srcs[m], sem_i[m]).wait()
            pltpu.make_async_copy(dst_hbm.at[pl.ds(base, CHUNK)], dsts[m], sem_i[m]).wait()

    def issue_gather(j, m, p):
        @pl.when(_ok(j))
        def _():
            pltpu.async_copy(h_hbm.at[srcs[m]], rows_s[m], sem_g[p])
            pltpu.async_copy(h_hbm.at[dsts[m]], rows_d[m], sem_g[p])

    def wait_gather(j, m, p):
        @pl.when(_ok(j))
        def _():
            pltpu.make_async_copy(h_hbm.at[srcs[m]], rows_s[m], sem_g[p]).wait()
            pltpu.make_async_copy(h_hbm.at[dsts[m]], rows_d[m], sem_g[p]).wait()

    def compute(j, m):
        @pl.when(_ok(j))
        def _():
            def _bf16_word(x_ref, r, g):
                a = x_ref[r, pl.ds(g * 32, 16)]
                b = x_ref[r, pl.ds(g * 32 + 16, 16)]
                ra = a + 0x7FFF + ((a >> 16) & 1)
                rb = b + 0x7FFF + ((b >> 16) & 1)
                return ((ra >> 16) & 0xFFFF) | (rb & jnp.int32(-65536))

            def _row(r, rc):
                for g in range(4):
                    bf_p[m][r, pl.ds(g * 16, 16)] = _bf16_word(rows_s[m], r, g)
                    bf_p[m][r, pl.ds(H2 + g * 16, 16)] = _bf16_word(rows_d[m], r, g)
                return rc
            lax.fori_loop(0, CHUNK, _row, 0)

    def issue_write(j, m, p):
        @pl.when(_ok(j))
        def _():
            base = _cid(j) * CHUNK
            pltpu.async_copy(bf_p[m], hp_hbm.at[pl.ds(base, CHUNK)], sem_w[p])

    def wait_write(j, m, p):
        @pl.when(_ok(j))
        def _():
            base = _cid(j) * CHUNK
            pltpu.make_async_copy(bf_p[m], hp_hbm.at[pl.ds(base, CHUNK)], sem_w[p]).wait()

    issue_idx(0, 0)
    issue_idx(1, 1)
    wait_idx(0, 0)
    issue_gather(0, 0, 0)

    def _outer(t, carry):
        for r in range(4):
            k = 4 * t + r
            wait_write(k - 2, (r + 2) % 4, r % 2)
            wait_idx(k + 1, (r + 1) % 4)
            issue_gather(k + 1, (r + 1) % 4, (r + 1) % 2)
            issue_idx(k + 2, (r + 2) % 4)
            wait_gather(k, r % 4, r % 2)
            compute(k, r % 4)
            issue_write(k, r % 4, r % 2)
        return carry

    lax.fori_loop(0, CLOOP // 4, _outer, 0)
    wait_write(CLOOP - 2, (CLOOP - 2) % 4, CLOOP % 2)
    wait_write(CLOOP - 1, (CLOOP - 1) % 4, (CLOOP + 1) % 2)


_sc_pair = functools.partial(
    pl.kernel,
    out_type=jax.ShapeDtypeStruct((E, H), jnp.int32),
    mesh=_SC_MESH,
    scratch_types=(
        [pltpu.VMEM((CHUNK,), jnp.int32)] * 8
        + [pltpu.VMEM((CHUNK, H), jnp.int32)] * 12
        + [pltpu.SemaphoreType.DMA] * 8
    ),
)(_sc_pair_body)


def _mlp2_kern(x_ref, w1_ref, b1_ref, w2_ref, b2_ref, o_ref):
    t = jnp.maximum(
        jnp.dot(x_ref[...].astype(w1_ref.dtype), w1_ref[...],
                preferred_element_type=jnp.float32)
        + b1_ref[...], 0.0)
    o_ref[...] = (jnp.dot(t.astype(w2_ref.dtype), w2_ref[...],
                          preferred_element_type=jnp.float32)
                  + b2_ref[...])


def _mlp2(x, w1, b1, w2, b2, blk):
    m, din = x.shape
    dmid = w1.shape[1]
    dout = w2.shape[1]
    grid = m // blk
    return pl.pallas_call(
        _mlp2_kern,
        grid=(grid,),
        in_specs=[
            pl.BlockSpec((blk, din), lambda i: (i, 0)),
            pl.BlockSpec((din, dmid), lambda i: (0, 0)),
            pl.BlockSpec((1, dmid), lambda i: (0, 0)),
            pl.BlockSpec((dmid, dout), lambda i: (0, 0)),
            pl.BlockSpec((1, dout), lambda i: (0, 0)),
        ],
        out_specs=pl.BlockSpec((blk, dout), lambda i: (i, 0)),
        out_shape=jax.ShapeDtypeStruct((m, dout), jnp.float32),
    )(x, w1, b1.reshape(1, -1), w2, b2.reshape(1, -1))


def _bond_mlp_kern(x_ref, w1_ref, b1_ref, w2_ref, b2_ref, o_ref):
    for j in range(8):
        xj = x_ref[:, j * 16:(j + 1) * 16].astype(w1_ref.dtype)
        t = jnp.maximum(
            jnp.dot(xj, w1_ref[...], preferred_element_type=jnp.float32)
            + b1_ref[...], 0.0)
        o_ref[:, j * H:(j + 1) * H] = (
            jnp.dot(t.astype(w2_ref.dtype), w2_ref[...],
                    preferred_element_type=jnp.float32) + b2_ref[...])


def _bond_mlp(ea8, w1, b1, w2, b2, blk=2000):
    m = ea8.shape[0]
    grid = m // blk
    return pl.pallas_call(
        _bond_mlp_kern,
        grid=(grid,),
        in_specs=[
            pl.BlockSpec((blk, 128), lambda i: (i, 0)),
            pl.BlockSpec((16, 256), lambda i: (0, 0)),
            pl.BlockSpec((1, 256), lambda i: (0, 0)),
            pl.BlockSpec((256, H), lambda i: (0, 0)),
            pl.BlockSpec((1, H), lambda i: (0, 0)),
        ],
        out_specs=pl.BlockSpec((blk, 8 * H), lambda i: (i, 0)),
        out_shape=jax.ShapeDtypeStruct((m, 8 * H), jnp.float32),
    )(ea8, w1, b1.reshape(1, -1), w2, b2.reshape(1, -1))


def _bond_mlp_kern(xt_ref, w1_ref, b1_ref, w2_ref, b2_ref, o_ref):
    t = jnp.maximum(
        jax.lax.dot_general(
            xt_ref[...].astype(w1_ref.dtype), w1_ref[...],
            (((0,), (0,)), ((), ())), preferred_element_type=jnp.float32)
        + b1_ref[...], 0.0)
    o_ref[...] = (jnp.dot(t.astype(w2_ref.dtype), w2_ref[...],
                          preferred_element_type=jnp.float32) + b2_ref[...])


def _bond_mlp(eat, w1, b1, w2, b2, blk=16000):
    grid = E // blk
    din = eat.shape[0]
    dmid = w1.shape[1]
    dout = w2.shape[1]
    return pl.pallas_call(
        _bond_mlp_kern,
        grid=(grid,),
        in_specs=[
            pl.BlockSpec((din, blk), lambda i: (0, i)),
            pl.BlockSpec((din, dmid), lambda i: (0, 0)),
            pl.BlockSpec((1, dmid), lambda i: (0, 0)),
            pl.BlockSpec((dmid, dout), lambda i: (0, 0)),
            pl.BlockSpec((1, dout), lambda i: (0, 0)),
        ],
        out_specs=pl.BlockSpec((blk, dout), lambda i: (i, 0)),
        out_shape=jax.ShapeDtypeStruct((E, dout), jnp.float32),
    )(eat, w1, b1.reshape(1, -1), w2, b2.reshape(1, -1))


def _gru_kern(p_ref, h_ref, wih_ref, whh_ref, bg_ref, o_ref):
    msg = p_ref[0] + p_ref[1]
    gi = jnp.dot(msg.astype(wih_ref.dtype), wih_ref[...],
                 preferred_element_type=jnp.float32) + bg_ref[...]
    gh = jnp.dot(h_ref[...].astype(whh_ref.dtype), whh_ref[...],
                 preferred_element_type=jnp.float32)
    z = jax.nn.sigmoid(gi[:, :H] + gh[:, :H])
    r = jax.nn.sigmoid(gi[:, H:2 * H] + gh[:, H:2 * H])
    n = jnp.tanh(gi[:, 2 * H:] + r * gh[:, 2 * H:])
    o_ref[...] = (1.0 - z) * n + z * h_ref[...]


def _gru(partials, h, wih, whh, bg, blk=2000):
    grid = N // blk
    return pl.pallas_call(
        _gru_kern,
        grid=(grid,),
        in_specs=[
            pl.BlockSpec((2, blk, H), lambda i: (0, i, 0)),
            pl.BlockSpec((blk, H), lambda i: (i, 0)),
            pl.BlockSpec((H, 3 * H), lambda i: (0, 0)),
            pl.BlockSpec((H, 3 * H), lambda i: (0, 0)),
            pl.BlockSpec((1, 3 * H), lambda i: (0, 0)),
        ],
        out_specs=pl.BlockSpec((blk, H), lambda i: (i, 0)),
        out_shape=jax.ShapeDtypeStruct((N, H), jnp.float32),
    )(partials, h, wih, whh, bg.reshape(1, -1))


def _bond_out_kern(hp_ref, w1_ref, b1_ref,
                   w2_ref, b2_ref, w3_ref, b3_ref, o_ref):
    bf16 = jnp.bfloat16
    hp = hp_ref[...]

    def _unpk(x, idx):
        w = pltpu.unpack_elementwise(
            x, index=idx,
            packed_dtype=jnp.int16, unpacked_dtype=jnp.int32)
        return pltpu.bitcast(w << 16, jnp.float32).astype(bf16)

    xs, xd = hp[:, :H2], hp[:, H2:]
    ys = jnp.concatenate(
        [_unpk(xs, 0), _unpk(xs, 1), _unpk(xd, 0), _unpk(xd, 1)], axis=1)
    t = jnp.maximum(
        jnp.dot(ys, w1_ref[...], preferred_element_type=jnp.float32)
        + b1_ref[...], 0.0)
    t = jnp.maximum(
        jnp.dot(t.astype(w2_ref.dtype), w2_ref[...],
                preferred_element_type=jnp.float32)
        + b2_ref[...], 0.0)
    o_ref[...] = (jax.lax.dot_general(
        w3_ref[...], t.astype(w3_ref.dtype), (((0,), (1,)), ((), ())),
        preferred_element_type=jnp.float32) + b3_ref[...])


def _bond_out(hp, wo1, bo1, wo2, bo2, wo3, bo3, blk=16000):
    grid = E // blk
    dmid = wo1.shape[1]
    dmid2 = wo2.shape[1]
    k = 8
    lo = jnp.array(_LO_IDX)
    hi = jnp.array(_HI_IDX)
    w1 = jnp.concatenate(
        [wo1[:H][lo], wo1[:H][hi], wo1[H:][lo], wo1[H:][hi]], axis=0)
    return pl.pallas_call(
        _bond_out_kern,
        grid=(grid,),
        in_specs=[
            pl.BlockSpec((blk, H), lambda i: (i, 0)),
            pl.BlockSpec((2 * H, dmid), lambda i: (0, 0)),
            pl.BlockSpec((1, dmid), lambda i: (0, 0)),
            pl.BlockSpec((dmid, dmid2), lambda i: (0, 0)),
            pl.BlockSpec((1, dmid2), lambda i: (0, 0)),
            pl.BlockSpec((dmid2, k), lambda i: (0, 0)),
            pl.BlockSpec((k, 1), lambda i: (0, 0)),
        ],
        out_specs=pl.BlockSpec((k, blk), lambda i: (0, i)),
        out_shape=jax.ShapeDtypeStruct((k, E), jnp.float32),
    )(hp, w1,
      bo1.reshape(1, -1), wo2, bo2.reshape(1, -1),
      jnp.pad(wo3, ((0, 0), (0, 8 - wo3.shape[1]))),
      jnp.pad(bo3, (0, 8 - bo3.shape[0])).reshape(-1, 1))


def _graph_out_kern(h_ref, w1_ref, b1_ref, w2_ref, b2_ref, w3_ref, b3_ref,
                    o_ref):
    g = jnp.sum(h_ref[...], axis=0, keepdims=True) * (1.0 / N)
    t = jnp.maximum(
        jnp.dot(g, w1_ref[...], preferred_element_type=jnp.float32)
        + b1_ref[...], 0.0)
    t = jnp.maximum(
        jnp.dot(t, w2_ref[...], preferred_element_type=jnp.float32)
        + b2_ref[...], 0.0)
    o_ref[...] = (jnp.dot(t, w3_ref[...], preferred_element_type=jnp.float32)
                  + b3_ref[...])


def _graph_out(h, wq1, bq1, wq2, bq2, wq3, bq3):
    return pl.pallas_call(
        _graph_out_kern,
        out_shape=jax.ShapeDtypeStruct((1, wq3.shape[1]), jnp.float32),
    )(h, wq1, bq1.reshape(1, -1), wq2, bq2.reshape(1, -1), wq3,
      bq3.reshape(1, -1))


def kernel(x, edge_index, edge_attr, Wa1, ba1, Wa2, ba2, Wb1, bb1, Wb2, bb2,
           Wgih, Wghh, bg, Wo1, bo1, Wo2, bo2, Wo3, bo3, Wq1, bq1, Wq2, bq2,
           Wq3, bq3):
    src = edge_index[0]
    dst = edge_index[1]

    wa2p = jnp.pad(Wa2, ((0, 0), (0, 1)))
    ba2p = jnp.pad(ba2, (0, 1))
    h = _mlp2(x, Wa1, ba1, wa2p, ba2p, blk=2000)

    bf16 = jnp.bfloat16
    hb = _bond_mlp(edge_attr.T, Wb1.astype(bf16), bb1, Wb2.astype(bf16), bb2)

    zrows = jnp.zeros((ROWS_PER_SUB, H), jnp.float32)
    for _ in range(3):
        partials = _sc_msg(h, hb, src, dst, zrows)
        h = _gru(partials, h, Wgih.astype(bf16), Wghh.astype(bf16), bg)

    hp = _sc_pair(jax.lax.bitcast_convert_type(h, jnp.int32), src, dst)
    bond_t = _bond_out(hp, Wo1.astype(bf16), bo1, Wo2.astype(bf16),
                       bo2, Wo3.astype(bf16), bo3)
    bond_scores = bond_t[:Wo3.shape[1]].T

    graph_scores = _graph_out(h, Wq1, bq1, Wq2, bq2, Wq3, bq3)
    return bond_scores, graph_scores.reshape(-1)


# speedup vs baseline: 1.0467x; 1.0289x over previous
import functools

import jax
import jax.numpy as jnp
from jax import lax
from jax.experimental import pallas as pl
from jax.experimental.pallas import tpu as pltpu
from jax.experimental.pallas import tpu_sc as plsc

N = 10000
E = 160000
H = 128
CHUNK = 64
NTILES = 32
NCHUNKS = E // CHUNK
CLOOP = 80
NPAD = 10112
ROWS_PER_SUB = NPAD // 16

_SC_MESH = plsc.VectorSubcoreMesh(core_axis_name="c", subcore_axis_name="s")

H2 = H // 2

_LO_IDX = tuple(32 * (j // 16) + j % 16 for j in range(H2))
_HI_IDX = tuple(32 * (j // 16) + 16 + j % 16 for j in range(H2))


def _sc_msg_body(h_hbm, hb_hbm, src_hbm, dst_hbm, zero_hbm, out_hbm,
                 s0, s1, s2, s3, d0, d1, d2, d3, r0, r1, r2, r3, b0, b1,
                 acc_sh,
                 si0, si1, si2, si3, sd0, sd1, sd2, sd3,
                 sg0, sg1, sg2, sg3, sh0, sh1, ss0, ss1):
    srcs = [s0, s1, s2, s3]
    dsts = [d0, d1, d2, d3]
    rows = [r0, r1, r2, r3]
    hbb = [b0, b1]
    sem_i = [si0, si1, si2, si3]
    sem_di = [sd0, sd1, sd2, sd3]
    sem_g = [sg0, sg1, sg2, sg3]
    sem_h = [sh0, sh1]
    sem_s = [ss0, ss1]

    c = lax.axis_index("c")
    s = lax.axis_index("s")
    wid = s * 2 + c

    pltpu.sync_copy(zero_hbm, acc_sh.at[pl.ds(s * ROWS_PER_SUB, ROWS_PER_SUB)])
    plsc.subcore_barrier()

    def _cid(j):
        return wid + j * NTILES

    def _ok(j):
        return jnp.logical_and(j >= 0, _cid(j) < NCHUNKS)

    def issue_src_idx(j, m):
        @pl.when(_ok(j))
        def _():
            base = _cid(j) * CHUNK
            pltpu.async_copy(src_hbm.at[pl.ds(base, CHUNK)], srcs[m], sem_i[m])

    def wait_src_idx(j, m):
        @pl.when(_ok(j))
        def _():
            base = _cid(j) * CHUNK
            pltpu.make_async_copy(src_hbm.at[pl.ds(base, CHUNK)], srcs[m], sem_i[m]).wait()

    def issue_dst_idx(j, m):
        @pl.when(_ok(j))
        def _():
            base = _cid(j) * CHUNK
            pltpu.async_copy(dst_hbm.at[pl.ds(base, CHUNK)], dsts[m], sem_di[m])

    def wait_dst_idx(j, m):
        @pl.when(_ok(j))
        def _():
            base = _cid(j) * CHUNK
            pltpu.make_async_copy(dst_hbm.at[pl.ds(base, CHUNK)], dsts[m], sem_di[m]).wait()

    def issue_gather(j, m, p):
        @pl.when(_ok(j))
        def _():
            pltpu.async_copy(h_hbm.at[srcs[m]], rows[m], sem_g[p])

    def wait_gather(j, m, p):
        @pl.when(_ok(j))
        def _():
            pltpu.make_async_copy(h_hbm.at[srcs[m]], rows[m], sem_g[p]).wait()

    def issue_hb(j, p):
        @pl.when(_ok(j))
        def _():
            base = _cid(j) * CHUNK
            pltpu.async_copy(hb_hbm.at[pl.ds(base, CHUNK)], hbb[p], sem_h[p])

    def wait_hb(j, p):
        @pl.when(_ok(j))
        def _():
            base = _cid(j) * CHUNK
            pltpu.make_async_copy(hb_hbm.at[pl.ds(base, CHUNK)], hbb[p], sem_h[p]).wait()

    def compute(j, m, p):
        @pl.when(_ok(j))
        def _():
            def _row(r, rc):
                for q in range(8):
                    sl = pl.ds(q * 16, 16)
                    v = rows[m][r, sl] + hbb[p][r, sl]
                    rows[m][r, sl] = jnp.maximum(v, 0.0)
                return rc
            lax.fori_loop(0, CHUNK, _row, 0)

    def issue_scatter(j, m, p):
        @pl.when(_ok(j))
        def _():
            pltpu.async_copy(rows[m], acc_sh.at[dsts[m]], sem_s[p], add=True)

    def wait_scatter(j, m, p):
        @pl.when(_ok(j))
        def _():
            pltpu.make_async_copy(rows[m], acc_sh.at[dsts[m]], sem_s[p]).wait()

    issue_src_idx(0, 0)
    issue_src_idx(1, 1)
    issue_src_idx(2, 2)
    wait_src_idx(0, 0)
    issue_gather(0, 0, 0)
    wait_src_idx(1, 1)
    issue_gather(1, 1, 1)
    issue_dst_idx(0, 0)
    issue_hb(0, 0)

    def _outer(t, carry):
        for r in range(4):
            k = 4 * t + r
            wait_scatter(k - 2, (r + 2) % 4, r % 2)
            wait_src_idx(k + 2, (r + 2) % 4)
            issue_gather(k + 2, (r + 2) % 4, (r + 2) % 4)
            issue_src_idx(k + 3, (r + 3) % 4)
            issue_dst_idx(k + 1, (r + 1) % 4)
            issue_hb(k + 1, (r + 1) % 2)
            wait_gather(k, r % 4, r % 4)
            wait_hb(k, r % 2)
            wait_dst_idx(k, r % 4)
            compute(k, r % 4, r % 2)
            issue_scatter(k, r % 4, r % 2)
        return carry

    lax.fori_loop(0, CLOOP // 4, _outer, 0)
    wait_scatter(CLOOP - 2, (CLOOP - 2) % 4, CLOOP % 2)
    wait_scatter(CLOOP - 1, (CLOOP - 1) % 4, (CLOOP + 1) % 2)

    plsc.subcore_barrier()
    pltpu.sync_copy(acc_sh.at[pl.ds(s * ROWS_PER_SUB, ROWS_PER_SUB)],
                    out_hbm.at[c, pl.ds(s * ROWS_PER_SUB, ROWS_PER_SUB)])


_sc_msg = functools.partial(
    pl.kernel,
    out_type=jax.ShapeDtypeStruct((2, NPAD, H), jnp.float32),
    mesh=_SC_MESH,
    scratch_types=(
        [pltpu.VMEM((CHUNK,), jnp.int32)] * 8
        + [pltpu.VMEM((CHUNK, H), jnp.float32)] * 6
        + [pltpu.VMEM_SHARED((NPAD, H), jnp.float32)]
        + [pltpu.SemaphoreType.DMA] * 16
    ),
)(_sc_msg_body)


def _sc_pair_body(h_hbm, src_hbm, dst_hbm, hp_hbm,
                  s0, s1, s2, s3, d0, d1, d2, d3,
                  rs0, rs1, rs2, rs3, rd0, rd1, rd2, rd3,
                  bp0, bp1, bp2, bp3,
                  si0, si1, si2, si3, sg0, sg1, sg2, sg3, sw0, sw1):
    srcs = [s0, s1, s2, s3]
    dsts = [d0, d1, d2, d3]
    rows_s = [rs0, rs1, rs2, rs3]
    rows_d = [rd0, rd1, rd2, rd3]
    bf_p = [bp0, bp1, bp2, bp3]
    sem_i = [si0, si1, si2, si3]
    sem_g = [sg0, sg1, sg2, sg3]
    sem_w = [sw0, sw1]

    c = lax.axis_index("c")
    s = lax.axis_index("s")
    wid = s * 2 + c

    def _cid(j):
        return wid + j * NTILES

    def _ok(j):
        return jnp.logical_and(j >= 0, _cid(j) < NCHUNKS)

    def issue_idx(j, m):
        @pl.when(_ok(j))
        def _():
            base = _cid(j) * CHUNK
            pltpu.async_copy(src_hbm.at[pl.ds(base, CHUNK)], srcs[m], sem_i[m])
            pltpu.async_copy(dst_hbm.at[pl.ds(base, CHUNK)], dsts[m], sem_i[m])

    def wait_idx(j, m):
        @pl.when(_ok(j))
        def _():
            base = _cid(j) * CHUNK
            pltpu.make_async_copy(src_hbm.at[pl.ds(base, CHUNK)], srcs[m], sem_i[m]).wait()
            pltpu.make_async_copy(dst_hbm.at[pl.ds(base, CHUNK)], dsts[m], sem_i[m]).wait()

    def issue_gather(j, m, p):
        @pl.when(_ok(j))
        def _():
            pltpu.async_copy(h_hbm.at[srcs[m]], rows_s[m], sem_g[p])
            pltpu.async_copy(h_hbm.at[dsts[m]], rows_d[m], sem_g[p])

    def wait_gather(j, m, p):
        @pl.when(_ok(j))
        def _():
            pltpu.make_async_copy(h_hbm.at[srcs[m]], rows_s[m], sem_g[p]).wait()
            pltpu.make_async_copy(h_hbm.at[dsts[m]], rows_d[m], sem_g[p]).wait()

    def compute(j, m):
        @pl.when(_ok(j))
        def _():
            def _bf16_word(x_ref, r, g):
                a = x_ref[r, pl.ds(g * 32, 16)]
                b = x_ref[r, pl.ds(g * 32 + 16, 16)]
                ra = a + 0x7FFF + ((a >> 16) & 1)
                rb = b + 0x7FFF + ((b >> 16) & 1)
                return ((ra >> 16) & 0xFFFF) | (rb & jnp.int32(-65536))

            def _row(r, rc):
                for g in range(4):
                    bf_p[m][r, pl.ds(g * 16, 16)] = _bf16_word(rows_s[m], r, g)
                    bf_p[m][r, pl.ds(H2 + g * 16, 16)] = _bf16_word(rows_d[m], r, g)
                return rc
            lax.fori_loop(0, CHUNK, _row, 0)

    def issue_write(j, m, p):
        @pl.when(_ok(j))
        def _():
            base = _cid(j) * CHUNK
            pltpu.async_copy(bf_p[m], hp_hbm.at[pl.ds(base, CHUNK)], sem_w[p])

    def wait_write(j, m, p):
        @pl.when(_ok(j))
        def _():
            base = _cid(j) * CHUNK
            pltpu.make_async_copy(bf_p[m], hp_hbm.at[pl.ds(base, CHUNK)], sem_w[p]).wait()

    issue_idx(0, 0)
    issue_idx(1, 1)
    issue_idx(2, 2)
    wait_idx(0, 0)
    issue_gather(0, 0, 0)
    wait_idx(1, 1)
    issue_gather(1, 1, 1)

    def _outer(t, carry):
        for r in range(4):
            k = 4 * t + r
            wait_write(k - 2, (r + 2) % 4, r % 2)
            wait_idx(k + 2, (r + 2) % 4)
            issue_gather(k + 2, (r + 2) % 4, (r + 2) % 4)
            issue_idx(k + 3, (r + 3) % 4)
            wait_gather(k, r % 4, r % 4)
            compute(k, r % 4)
            issue_write(k, r % 4, r % 2)
        return carry

    lax.fori_loop(0, CLOOP // 4, _outer, 0)
    wait_write(CLOOP - 2, (CLOOP - 2) % 4, CLOOP % 2)
    wait_write(CLOOP - 1, (CLOOP - 1) % 4, (CLOOP + 1) % 2)


_sc_pair = functools.partial(
    pl.kernel,
    out_type=jax.ShapeDtypeStruct((E, H), jnp.int32),
    mesh=_SC_MESH,
    scratch_types=(
        [pltpu.VMEM((CHUNK,), jnp.int32)] * 8
        + [pltpu.VMEM((CHUNK, H), jnp.int32)] * 12
        + [pltpu.SemaphoreType.DMA] * 10
    ),
)(_sc_pair_body)


def _mlp2_kern(x_ref, w1_ref, b1_ref, w2_ref, b2_ref, o_ref):
    t = jnp.maximum(
        jnp.dot(x_ref[...].astype(w1_ref.dtype), w1_ref[...],
                preferred_element_type=jnp.float32)
        + b1_ref[...], 0.0)
    o_ref[...] = (jnp.dot(t.astype(w2_ref.dtype), w2_ref[...],
                          preferred_element_type=jnp.float32)
                  + b2_ref[...])


def _mlp2(x, w1, b1, w2, b2, blk):
    m, din = x.shape
    dmid = w1.shape[1]
    dout = w2.shape[1]
    grid = m // blk
    return pl.pallas_call(
        _mlp2_kern,
        grid=(grid,),
        in_specs=[
            pl.BlockSpec((blk, din), lambda i: (i, 0)),
            pl.BlockSpec((din, dmid), lambda i: (0, 0)),
            pl.BlockSpec((1, dmid), lambda i: (0, 0)),
            pl.BlockSpec((dmid, dout), lambda i: (0, 0)),
            pl.BlockSpec((1, dout), lambda i: (0, 0)),
        ],
        out_specs=pl.BlockSpec((blk, dout), lambda i: (i, 0)),
        out_shape=jax.ShapeDtypeStruct((m, dout), jnp.float32),
    )(x, w1, b1.reshape(1, -1), w2, b2.reshape(1, -1))


def _bond_mlp_kern(x_ref, w1_ref, b1_ref, w2_ref, b2_ref, o_ref):
    for j in range(8):
        xj = x_ref[:, j * 16:(j + 1) * 16].astype(w1_ref.dtype)
        t = jnp.maximum(
            jnp.dot(xj, w1_ref[...], preferred_element_type=jnp.float32)
            + b1_ref[...], 0.0)
        o_ref[:, j * H:(j + 1) * H] = (
            jnp.dot(t.astype(w2_ref.dtype), w2_ref[...],
                    preferred_element_type=jnp.float32) + b2_ref[...])


def _bond_mlp(ea8, w1, b1, w2, b2, blk=2000):
    m = ea8.shape[0]
    grid = m // blk
    return pl.pallas_call(
        _bond_mlp_kern,
        grid=(grid,),
        in_specs=[
            pl.BlockSpec((blk, 128), lambda i: (i, 0)),
            pl.BlockSpec((16, 256), lambda i: (0, 0)),
            pl.BlockSpec((1, 256), lambda i: (0, 0)),
            pl.BlockSpec((256, H), lambda i: (0, 0)),
            pl.BlockSpec((1, H), lambda i: (0, 0)),
        ],
        out_specs=pl.BlockSpec((blk, 8 * H), lambda i: (i, 0)),
        out_shape=jax.ShapeDtypeStruct((m, 8 * H), jnp.float32),
    )(ea8, w1, b1.reshape(1, -1), w2, b2.reshape(1, -1))


def _bond_mlp_kern(xt_ref, w1_ref, b1_ref, w2_ref, b2_ref, o_ref):
    t = jnp.maximum(
        jax.lax.dot_general(
            xt_ref[...].astype(w1_ref.dtype), w1_ref[...],
            (((0,), (0,)), ((), ())), preferred_element_type=jnp.float32)
        + b1_ref[...], 0.0)
    o_ref[...] = (jnp.dot(t.astype(w2_ref.dtype), w2_ref[...],
                          preferred_element_type=jnp.float32) + b2_ref[...])


def _bond_mlp(eat, w1, b1, w2, b2, blk=16000):
    grid = E // blk
    din = eat.shape[0]
    dmid = w1.shape[1]
    dout = w2.shape[1]
    return pl.pallas_call(
        _bond_mlp_kern,
        grid=(grid,),
        in_specs=[
            pl.BlockSpec((din, blk), lambda i: (0, i)),
            pl.BlockSpec((din, dmid), lambda i: (0, 0)),
            pl.BlockSpec((1, dmid), lambda i: (0, 0)),
            pl.BlockSpec((dmid, dout), lambda i: (0, 0)),
            pl.BlockSpec((1, dout), lambda i: (0, 0)),
        ],
        out_specs=pl.BlockSpec((blk, dout), lambda i: (i, 0)),
        out_shape=jax.ShapeDtypeStruct((E, dout), jnp.float32),
    )(eat, w1, b1.reshape(1, -1), w2, b2.reshape(1, -1))


def _gru_kern(p_ref, h_ref, wih_ref, whh_ref, bg_ref, o_ref):
    msg = p_ref[0] + p_ref[1]
    gi = jnp.dot(msg.astype(wih_ref.dtype), wih_ref[...],
                 preferred_element_type=jnp.float32) + bg_ref[...]
    gh = jnp.dot(h_ref[...].astype(whh_ref.dtype), whh_ref[...],
                 preferred_element_type=jnp.float32)
    z = jax.nn.sigmoid(gi[:, :H] + gh[:, :H])
    r = jax.nn.sigmoid(gi[:, H:2 * H] + gh[:, H:2 * H])
    n = jnp.tanh(gi[:, 2 * H:] + r * gh[:, 2 * H:])
    o_ref[...] = (1.0 - z) * n + z * h_ref[...]


def _gru(partials, h, wih, whh, bg, blk=2000):
    grid = N // blk
    return pl.pallas_call(
        _gru_kern,
        grid=(grid,),
        in_specs=[
            pl.BlockSpec((2, blk, H), lambda i: (0, i, 0)),
            pl.BlockSpec((blk, H), lambda i: (i, 0)),
            pl.BlockSpec((H, 3 * H), lambda i: (0, 0)),
            pl.BlockSpec((H, 3 * H), lambda i: (0, 0)),
            pl.BlockSpec((1, 3 * H), lambda i: (0, 0)),
        ],
        out_specs=pl.BlockSpec((blk, H), lambda i: (i, 0)),
        out_shape=jax.ShapeDtypeStruct((N, H), jnp.float32),
    )(partials, h, wih, whh, bg.reshape(1, -1))


def _bond_out_kern(hp_ref, w1_ref, b1_ref,
                   w2_ref, b2_ref, w3_ref, b3_ref, o_ref):
    bf16 = jnp.bfloat16
    hp = hp_ref[...]

    def _unpk(x, idx):
        w = pltpu.unpack_elementwise(
            x, index=idx,
            packed_dtype=jnp.int16, unpacked_dtype=jnp.int32)
        return pltpu.bitcast(w << 16, jnp.float32).astype(bf16)

    xs, xd = hp[:, :H2], hp[:, H2:]
    ys = jnp.concatenate(
        [_unpk(xs, 0), _unpk(xs, 1), _unpk(xd, 0), _unpk(xd, 1)], axis=1)
    t = jnp.maximum(
        jnp.dot(ys, w1_ref[...], preferred_element_type=jnp.float32)
        + b1_ref[...], 0.0)
    t = jnp.maximum(
        jnp.dot(t.astype(w2_ref.dtype), w2_ref[...],
                preferred_element_type=jnp.float32)
        + b2_ref[...], 0.0)
    o_ref[...] = (jax.lax.dot_general(
        w3_ref[...], t.astype(w3_ref.dtype), (((0,), (1,)), ((), ())),
        preferred_element_type=jnp.float32) + b3_ref[...])


def _bond_out(hp, wo1, bo1, wo2, bo2, wo3, bo3, blk=16000):
    grid = E // blk
    dmid = wo1.shape[1]
    dmid2 = wo2.shape[1]
    k = 8
    lo = jnp.array(_LO_IDX)
    hi = jnp.array(_HI_IDX)
    w1 = jnp.concatenate(
        [wo1[:H][lo], wo1[:H][hi], wo1[H:][lo], wo1[H:][hi]], axis=0)
    return pl.pallas_call(
        _bond_out_kern,
        grid=(grid,),
        in_specs=[
            pl.BlockSpec((blk, H), lambda i: (i, 0)),
            pl.BlockSpec((2 * H, dmid), lambda i: (0, 0)),
            pl.BlockSpec((1, dmid), lambda i: (0, 0)),
            pl.BlockSpec((dmid, dmid2), lambda i: (0, 0)),
            pl.BlockSpec((1, dmid2), lambda i: (0, 0)),
            pl.BlockSpec((dmid2, k), lambda i: (0, 0)),
            pl.BlockSpec((k, 1), lambda i: (0, 0)),
        ],
        out_specs=pl.BlockSpec((k, blk), lambda i: (0, i)),
        out_shape=jax.ShapeDtypeStruct((k, E), jnp.float32),
    )(hp, w1,
      bo1.reshape(1, -1), wo2, bo2.reshape(1, -1),
      jnp.pad(wo3, ((0, 0), (0, 8 - wo3.shape[1]))),
      jnp.pad(bo3, (0, 8 - bo3.shape[0])).reshape(-1, 1))


def _graph_out_kern(h_ref, w1_ref, b1_ref, w2_ref, b2_ref, w3_ref, b3_ref,
                    o_ref):
    g = jnp.sum(h_ref[...], axis=0, keepdims=True) * (1.0 / N)
    t = jnp.maximum(
        jnp.dot(g, w1_ref[...], preferred_element_type=jnp.float32)
        + b1_ref[...], 0.0)
    t = jnp.maximum(
        jnp.dot(t, w2_ref[...], preferred_element_type=jnp.float32)
        + b2_ref[...], 0.0)
    o_ref[...] = (jnp.dot(t, w3_ref[...], preferred_element_type=jnp.float32)
                  + b3_ref[...])


def _graph_out(h, wq1, bq1, wq2, bq2, wq3, bq3):
    return pl.pallas_call(
        _graph_out_kern,
        out_shape=jax.ShapeDtypeStruct((1, wq3.shape[1]), jnp.float32),
    )(h, wq1, bq1.reshape(1, -1), wq2, bq2.reshape(1, -1), wq3,
      bq3.reshape(1, -1))


def kernel(x, edge_index, edge_attr, Wa1, ba1, Wa2, ba2, Wb1, bb1, Wb2, bb2,
           Wgih, Wghh, bg, Wo1, bo1, Wo2, bo2, Wo3, bo3, Wq1, bq1, Wq2, bq2,
           Wq3, bq3):
    src = edge_index[0]
    dst = edge_index[1]

    wa2p = jnp.pad(Wa2, ((0, 0), (0, 1)))
    ba2p = jnp.pad(ba2, (0, 1))
    h = _mlp2(x, Wa1, ba1, wa2p, ba2p, blk=2000)

    bf16 = jnp.bfloat16
    hb = _bond_mlp(edge_attr.T, Wb1.astype(bf16), bb1, Wb2.astype(bf16), bb2)

    zrows = jnp.zeros((ROWS_PER_SUB, H), jnp.float32)
    for _ in range(3):
        partials = _sc_msg(h, hb, src, dst, zrows)
        h = _gru(partials, h, Wgih.astype(bf16), Wghh.astype(bf16), bg)

    hp = _sc_pair(jax.lax.bitcast_convert_type(h, jnp.int32), src, dst)
    bond_t = _bond_out(hp, Wo1.astype(bf16), bo1, Wo2.astype(bf16),
                       bo2, Wo3.astype(bf16), bo3)
    bond_scores = bond_t[:Wo3.shape[1]].T

    graph_scores = _graph_out(h, Wq1, bq1, Wq2, bq2, Wq3, bq3)
    return bond_scores, graph_scores.reshape(-1)
